# Initial kernel scaffold; baseline (speedup 1.0000x reference)
import jax
import jax.numpy as jnp
from jax.experimental import pallas as pl


def kernel(x, edge_index, edge_attr, params):
    raise NotImplementedError("write your pallas kernel here")


# speedup vs baseline: 2.6276x; 2.6276x over previous
import functools

import jax
import jax.numpy as jnp
from jax import lax
from jax.experimental import pallas as pl
from jax.experimental.pallas import tpu as pltpu
from jax.experimental.pallas import tpu_sc as plsc

_N = 10000
_E = 160000
_DIN = 128
_H = 16
_EDIM = 16
_NCLS = 16
_ITERS = 4
_EPS = 1e-5

_NW = 32
_CHUNK = 128
_EPW = 5120
_NCHUNK = _EPW // _CHUNK
_E_PAD = _NW * _EPW
_N_PAD = 10240
_RPT = _N_PAD // 16
_GRP = 8

_FNET_T = 2048
_BMM_T = 2048


def _stats_body(ea_ref, s_ref, s2_ref):
    x = ea_ref[...]
    s_ref[...] = jnp.sum(x, axis=0, keepdims=True)
    s2_ref[...] = jnp.sum(x * x, axis=0, keepdims=True)


def _stats_call(ea):
    s, s2 = pl.pallas_call(
        _stats_body,
        out_shape=(jax.ShapeDtypeStruct((1, 128), jnp.float32),
                   jax.ShapeDtypeStruct((1, 128), jnp.float32)),
    )(ea.reshape(_E // 8, 128))
    m = s.reshape(8, _EDIM).sum(axis=0) / _E
    m2 = s2.reshape(8, _EDIM).sum(axis=0) / _E
    return m, m2 - m * m


def _fnet_body(e_ref, w1_ref, b1_ref, w2_ref, b2_ref, w3_ref, b3_ref, out_ref):
    h1 = jnp.maximum(jnp.dot(e_ref[...], w1_ref[...]) + b1_ref[...], 0.0)
    h2 = jnp.maximum(jnp.dot(h1, w2_ref[...]) + b2_ref[...], 0.0)
    out_ref[...] = jnp.dot(h2, w3_ref[...]) + b3_ref[...]


def _fnet_call(ea_pad, w1, b1, w2, b2, w3, b3):
    g = _E_PAD // _FNET_T
    fixed = lambda i: (0, 0)
    return pl.pallas_call(
        _fnet_body,
        grid=(g,),
        in_specs=[
            pl.BlockSpec((_FNET_T, _EDIM), lambda i: (i, 0)),
            pl.BlockSpec((_EDIM, 32), fixed),
            pl.BlockSpec((1, 32), fixed),
            pl.BlockSpec((32, 128), fixed),
            pl.BlockSpec((1, 128), fixed),
            pl.BlockSpec((128, _H * _H), fixed),
            pl.BlockSpec((1, _H * _H), fixed),
        ],
        out_specs=pl.BlockSpec((_FNET_T, _H * _H), lambda i: (i, 0)),
        out_shape=jax.ShapeDtypeStruct((_E_PAD, _H * _H), jnp.float32),
    )(ea_pad, w1, b1, w2, b2, w3, b3)


def _encode_body(x_ref, g1_ref, b1_ref, ew_ref, eb_ref, g2_ref, b2_ref, out_ref):
    x = x_ref[...]
    m = jnp.mean(x, axis=0, keepdims=True)
    v = jnp.mean((x - m) * (x - m), axis=0, keepdims=True)
    h = (x - m) * lax.rsqrt(v + _EPS) * g1_ref[...] + b1_ref[...]
    h = jnp.dot(h, ew_ref[...]) + eb_ref[...]
    m2 = jnp.mean(h, axis=0, keepdims=True)
    v2 = jnp.mean((h - m2) * (h - m2), axis=0, keepdims=True)
    out_ref[...] = jnp.maximum(
        (h - m2) * lax.rsqrt(v2 + _EPS) * g2_ref[...] + b2_ref[...], 0.0)


def _encode_call(x, g1, b1, ew, eb, g2, b2):
    return pl.pallas_call(
        _encode_body,
        out_shape=jax.ShapeDtypeStruct((_N, _H), jnp.float32),
    )(x, g1, b1, ew, eb, g2, b2)


def _bmm_body(xj_ref, w_ref, out_ref):
    xj = xj_ref[...]
    w = w_ref[...]
    ri = lax.broadcasted_iota(jnp.int32, (_H, _H * _H), 0)
    ci = lax.broadcasted_iota(jnp.int32, (_H, _H * _H), 1)
    rep = (ci // _H == ri).astype(jnp.float32)
    si = lax.broadcasted_iota(jnp.int32, (_H * _H, _H), 0)
    so = lax.broadcasted_iota(jnp.int32, (_H * _H, _H), 1)
    ssum = (si % _H == so).astype(jnp.float32)
    xrep = jnp.dot(xj, rep)
    out_ref[...] = jnp.dot(xrep * w, ssum)


def _bmm_call(xj, w):
    g = _E_PAD // _BMM_T
    return pl.pallas_call(
        _bmm_body,
        grid=(g,),
        in_specs=[
            pl.BlockSpec((_BMM_T, _H), lambda i: (i, 0)),
            pl.BlockSpec((_BMM_T, _H * _H), lambda i: (i, 0)),
        ],
        out_specs=pl.BlockSpec((_BMM_T, _H), lambda i: (i, 0)),
        out_shape=jax.ShapeDtypeStruct((_E_PAD, _H), jnp.float32),
    )(xj, w)


def _sig(t):
    return 1.0 / (1.0 + jnp.exp(-t))


def _ln(v, g, b):
    m = jnp.mean(v, axis=-1, keepdims=True)
    var = jnp.mean((v - m) * (v - m), axis=-1, keepdims=True)
    return (v - m) * lax.rsqrt(var + _EPS) * g + b


def _gru_body(a0_ref, a1_ref, c0_ref, c1_ref, h_ref, igw_ref, igb_ref,
              lig_ref, lib_ref, lhg_ref, lhb_ref, wih_ref, bih_ref,
              whh_ref, bhh_ref, eccb_ref, out_ref):
    s = (a0_ref[...] + a1_ref[...])[:_N]
    cnt = (c0_ref[...] + c1_ref[...])[:_N, 0:1]
    msg = s / jnp.maximum(cnt, 1.0) + eccb_ref[...]
    h = h_ref[...]
    gate = _sig(jnp.dot(h, igw_ref[...]) + igb_ref[...])
    gated = _ln(gate * msg, lig_ref[...], lib_ref[...])
    h2 = _ln(h, lhg_ref[...], lhb_ref[...])
    gi = jnp.dot(gated, wih_ref[...]) + bih_ref[...]
    gh = jnp.dot(h2, whh_ref[...]) + bhh_ref[...]
    r = _sig(gi[:, :_H] + gh[:, :_H])
    z = _sig(gi[:, _H:2 * _H] + gh[:, _H:2 * _H])
    n = jnp.tanh(gi[:, 2 * _H:] + r * gh[:, 2 * _H:])
    out_ref[...] = (1.0 - z) * n + z * h2


def _gru_call(a0, a1, c0, c1, h, igw, igb, lig, lib_, lhg, lhb,
              wih, bih, whh, bhh, eccb):
    return pl.pallas_call(
        _gru_body,
        out_shape=jax.ShapeDtypeStruct((_N, _H), jnp.float32),
    )(a0, a1, c0, c1, h, igw, igb, lig, lib_, lhg, lhb, wih, bih, whh, bhh, eccb)


def _cls_body(hc_ref, w_ref, b_ref, out_ref):
    out_ref[...] = jnp.dot(hc_ref[...], w_ref[...]) + b_ref[...]


def _cls_call(hc, w, b):
    return pl.pallas_call(
        _cls_body,
        out_shape=jax.ShapeDtypeStruct((_N, _NCLS), jnp.float32),
    )(hc, w, b)


@functools.lru_cache(maxsize=None)
def _gather_kernel():
    mesh = plsc.VectorSubcoreMesh(core_axis_name="c", subcore_axis_name="s")

    @functools.partial(
        pl.kernel,
        mesh=mesh,
        out_type=jax.ShapeDtypeStruct((_E_PAD, _H), jnp.float32),
        scratch_types=[
            pltpu.VMEM((_NCHUNK, _CHUNK), jnp.int32),
            pltpu.VMEM((_EPW, _H), jnp.float32),
            pltpu.SemaphoreType.DMA,
        ],
        compiler_params=pltpu.CompilerParams(use_tc_tiling_on_sc=False),
    )
    def gather(tbl_hbm, src_hbm, out_hbm, idx_v, rows_v, sem):
        wid = lax.axis_index("s") * 2 + lax.axis_index("c")
        pltpu.sync_copy(src_hbm.at[wid], idx_v)

        def group(g, carry):
            base = g * _GRP
            copies = [
                pltpu.async_copy(
                    tbl_hbm.at[idx_v.at[base + b]],
                    rows_v.at[pl.ds((base + b) * _CHUNK, _CHUNK)],
                    sem,
                )
                for b in range(_GRP)
            ]
            for cp in copies:
                cp.wait()
            return carry

        lax.fori_loop(0, _NCHUNK // _GRP, group, 0, unroll=False)
        pltpu.sync_copy(rows_v, out_hbm.at[pl.ds(wid * _EPW, _EPW)])

    return gather


@functools.lru_cache(maxsize=None)
def _scatter_kernel():
    mesh = plsc.VectorSubcoreMesh(core_axis_name="c", subcore_axis_name="s")

    @functools.partial(
        pl.kernel,
        mesh=mesh,
        out_type=jax.ShapeDtypeStruct((2, _N_PAD, _H), jnp.float32),
        scratch_types=[
            pltpu.VMEM((_NCHUNK, _CHUNK), jnp.int32),
            pltpu.VMEM((_EPW, _H), jnp.float32),
            pltpu.VMEM_SHARED((_N_PAD, _H), jnp.float32),
            pltpu.SemaphoreType.DMA,
        ],
        compiler_params=pltpu.CompilerParams(use_tc_tiling_on_sc=False),
    )
    def scatter(vals_hbm, dst_hbm, z_hbm, out_hbm, idx_v, vals_v, acc_sh, sem):
        cid = lax.axis_index("c")
        sid = lax.axis_index("s")
        wid = sid * 2 + cid
        rows = pl.ds(sid * _RPT, _RPT)
        pltpu.sync_copy(z_hbm.at[rows], acc_sh.at[rows])
        pltpu.sync_copy(dst_hbm.at[wid], idx_v)
        pltpu.sync_copy(vals_hbm.at[pl.ds(wid * _EPW, _EPW)], vals_v)
        plsc.subcore_barrier()

        def chunk(j, carry):
            pltpu.sync_copy(vals_v.at[pl.ds(j * _CHUNK, _CHUNK)],
                            acc_sh.at[idx_v.at[j]], add=True)
            return carry

        lax.fori_loop(0, _NCHUNK, chunk, 0, unroll=False)
        plsc.subcore_barrier()
        pltpu.sync_copy(acc_sh.at[rows], out_hbm.at[cid, rows])

    return scatter


def kernel(x, edge_index, edge_attr, params):
    p = params
    f32 = jnp.float32
    pad = _E_PAD - _E
    src = edge_index[0].astype(jnp.int32)
    dst = edge_index[1].astype(jnp.int32)
    src3 = jnp.concatenate([src, jnp.zeros((pad,), jnp.int32)]) \
        .reshape(_NW, _NCHUNK, _CHUNK)
    dst3 = jnp.concatenate([dst, jnp.full((pad,), _N, jnp.int32)]) \
        .reshape(_NW, _NCHUNK, _CHUNK)
    zrows = jnp.zeros((_N_PAD, _H), f32)
    ones_vals = jnp.ones((_E_PAD, _H), f32)

    mean, var = _stats_call(edge_attr)
    a = p['fbn_g'] * lax.rsqrt(var + _EPS)
    c0 = p['fbn_b'] - mean * a
    w1f = a[:, None] * p['f1_W']
    b1f = (p['f1_b'] + c0 @ p['f1_W']).reshape(1, 32)
    ea_pad = jnp.concatenate([edge_attr, jnp.zeros((pad, _EDIM), f32)])
    w = _fnet_call(ea_pad, w1f, b1f,
                   p['f2_W'], p['f2_b'].reshape(1, 128),
                   p['f3_W'], p['f3_b'].reshape(1, _H * _H))

    h = _encode_call(x, p['bn1_g'].reshape(1, _DIN), p['bn1_b'].reshape(1, _DIN),
                     p['enc_W'], p['enc_b'].reshape(1, _H),
                     p['bn2_g'].reshape(1, _H), p['bn2_b'].reshape(1, _H))

    cnt_acc = _scatter_kernel()(ones_vals, dst3, zrows)

    igw = p['ig_W']
    igb = p['ig_b'].reshape(1, _H)
    lig = p['lni_g'].reshape(1, _H)
    lib_ = p['lni_b'].reshape(1, _H)
    lhg = p['lnh_g'].reshape(1, _H)
    lhb = p['lnh_b'].reshape(1, _H)
    wih = p['gru_Wih'].T
    bih = p['gru_bih'].reshape(1, 3 * _H)
    whh = p['gru_Whh'].T
    bhh = p['gru_bhh'].reshape(1, 3 * _H)
    eccb = p['ecc_bias'].reshape(1, _H)

    history = [h]
    for _ in range(_ITERS):
        xj = _gather_kernel()(h, src3)
        msg = _bmm_call(xj, w)
        acc = _scatter_kernel()(msg, dst3, zrows)
        h = _gru_call(acc[0], acc[1], cnt_acc[0], cnt_acc[1], h,
                      igw, igb, lig, lib_, lhg, lhb, wih, bih, whh, bhh, eccb)
        history.append(h)

    hcat = jnp.concatenate(history, axis=1)
    return _cls_call(hcat, p['cls_W'], p['cls_b'].reshape(1, _NCLS))


# speedup vs baseline: 3.2848x; 1.2501x over previous
import functools

import jax
import jax.numpy as jnp
from jax import lax
from jax.experimental import pallas as pl
from jax.experimental.pallas import tpu as pltpu
from jax.experimental.pallas import tpu_sc as plsc

_N = 10000
_E = 160000
_DIN = 128
_H = 16
_EDIM = 16
_NCLS = 16
_ITERS = 4
_EPS = 1e-5

_NW = 32
_CHUNK = 128
_EPW = 5120
_NCHUNK = _EPW // _CHUNK
_E_PAD = _NW * _EPW
_N_PAD = 10240
_RPT = _N_PAD // 16
_GRP = 8

_FNET_T = 2048
_BMM_T = 2048


def _stats_body(ea_ref, s_ref, s2_ref):
    x = ea_ref[...]
    s_ref[...] = jnp.sum(x, axis=0, keepdims=True)
    s2_ref[...] = jnp.sum(x * x, axis=0, keepdims=True)


def _stats_call(ea):
    s, s2 = pl.pallas_call(
        _stats_body,
        out_shape=(jax.ShapeDtypeStruct((1, 128), jnp.float32),
                   jax.ShapeDtypeStruct((1, 128), jnp.float32)),
    )(ea.reshape(_E // 8, 128))
    m = s.reshape(8, _EDIM).sum(axis=0) / _E
    m2 = s2.reshape(8, _EDIM).sum(axis=0) / _E
    return m, m2 - m * m


def _fnet_body(e_ref, w1_ref, b1_ref, w2_ref, b2_ref, w3_ref, b3_ref, out_ref):
    h1 = jnp.maximum(jnp.dot(e_ref[...], w1_ref[...]) + b1_ref[...], 0.0)
    h2 = jnp.maximum(jnp.dot(h1, w2_ref[...]) + b2_ref[...], 0.0)
    out_ref[...] = jnp.dot(h2, w3_ref[...]) + b3_ref[...]


def _fnet_call(ea_pad, w1, b1, w2, b2, w3, b3):
    g = _E_PAD // _FNET_T
    fixed = lambda i: (0, 0)
    return pl.pallas_call(
        _fnet_body,
        grid=(g,),
        in_specs=[
            pl.BlockSpec((_FNET_T, _EDIM), lambda i: (i, 0)),
            pl.BlockSpec((_EDIM, 32), fixed),
            pl.BlockSpec((1, 32), fixed),
            pl.BlockSpec((32, 128), fixed),
            pl.BlockSpec((1, 128), fixed),
            pl.BlockSpec((128, _H * _H), fixed),
            pl.BlockSpec((1, _H * _H), fixed),
        ],
        out_specs=pl.BlockSpec((_FNET_T, _H * _H), lambda i: (i, 0)),
        out_shape=jax.ShapeDtypeStruct((_E_PAD, _H * _H), jnp.float32),
    )(ea_pad, w1, b1, w2, b2, w3, b3)


def _encode_body(x_ref, g1_ref, b1_ref, ew_ref, eb_ref, g2_ref, b2_ref, out_ref):
    x = x_ref[...]
    m = jnp.mean(x, axis=0, keepdims=True)
    v = jnp.mean((x - m) * (x - m), axis=0, keepdims=True)
    h = (x - m) * lax.rsqrt(v + _EPS) * g1_ref[...] + b1_ref[...]
    h = jnp.dot(h, ew_ref[...]) + eb_ref[...]
    m2 = jnp.mean(h, axis=0, keepdims=True)
    v2 = jnp.mean((h - m2) * (h - m2), axis=0, keepdims=True)
    out_ref[...] = jnp.maximum(
        (h - m2) * lax.rsqrt(v2 + _EPS) * g2_ref[...] + b2_ref[...], 0.0)


def _encode_call(x, g1, b1, ew, eb, g2, b2):
    return pl.pallas_call(
        _encode_body,
        out_shape=jax.ShapeDtypeStruct((_N, _H), jnp.float32),
    )(x, g1, b1, ew, eb, g2, b2)


def _bmm_body(xj_ref, w_ref, out_ref):
    xjp = xj_ref[...]
    wp = w_ref[...]
    ri = lax.broadcasted_iota(jnp.int32, (_H, _H * _H), 0)
    ci = lax.broadcasted_iota(jnp.int32, (_H, _H * _H), 1)
    rep = (ci // _H == ri).astype(jnp.float32)
    si = lax.broadcasted_iota(jnp.int32, (_H * _H, _H), 0)
    so = lax.broadcasted_iota(jnp.int32, (_H * _H, _H), 1)
    ssum = (si % _H == so).astype(jnp.float32)
    cols = []
    for s in range(8):
        xs = xjp[:, s * _H:(s + 1) * _H]
        ws = wp[:, s * 256:(s + 1) * 256]
        cols.append(jnp.dot(jnp.dot(xs, rep) * ws, ssum))
    out_ref[...] = jnp.concatenate(cols, axis=1)


def _bmm_call(xjp, wp):
    g = _E_PAD // _BMM_T
    t8 = _BMM_T // 8
    return pl.pallas_call(
        _bmm_body,
        grid=(g,),
        in_specs=[
            pl.BlockSpec((t8, 128), lambda i: (i, 0)),
            pl.BlockSpec((t8, 8 * _H * _H), lambda i: (i, 0)),
        ],
        out_specs=pl.BlockSpec((t8, 128), lambda i: (i, 0)),
        out_shape=jax.ShapeDtypeStruct((_E_PAD // 8, 128), jnp.float32),
    )(xjp, wp)


def _sig(t):
    return 1.0 / (1.0 + jnp.exp(-t))


def _ln(v, g, b):
    m = jnp.mean(v, axis=-1, keepdims=True)
    var = jnp.mean((v - m) * (v - m), axis=-1, keepdims=True)
    return (v - m) * lax.rsqrt(var + _EPS) * g + b


def _gru_body(a0_ref, a1_ref, c0_ref, c1_ref, h_ref, igw_ref, igb_ref,
              lig_ref, lib_ref, lhg_ref, lhb_ref, wih_ref, bih_ref,
              whh_ref, bhh_ref, eccb_ref, out_ref):
    s = (a0_ref[...] + a1_ref[...])[:_N]
    cnt = (c0_ref[...] + c1_ref[...])[:_N, 0:1]
    msg = s / jnp.maximum(cnt, 1.0) + eccb_ref[...]
    h = h_ref[...]
    gate = _sig(jnp.dot(h, igw_ref[...]) + igb_ref[...])
    gated = _ln(gate * msg, lig_ref[...], lib_ref[...])
    h2 = _ln(h, lhg_ref[...], lhb_ref[...])
    gi = jnp.dot(gated, wih_ref[...]) + bih_ref[...]
    gh = jnp.dot(h2, whh_ref[...]) + bhh_ref[...]
    r = _sig(gi[:, :_H] + gh[:, :_H])
    z = _sig(gi[:, _H:2 * _H] + gh[:, _H:2 * _H])
    n = jnp.tanh(gi[:, 2 * _H:] + r * gh[:, 2 * _H:])
    out_ref[...] = (1.0 - z) * n + z * h2


def _gru_call(a0, a1, c0, c1, h, igw, igb, lig, lib_, lhg, lhb,
              wih, bih, whh, bhh, eccb):
    return pl.pallas_call(
        _gru_body,
        out_shape=jax.ShapeDtypeStruct((_N, _H), jnp.float32),
    )(a0, a1, c0, c1, h, igw, igb, lig, lib_, lhg, lhb, wih, bih, whh, bhh, eccb)


def _cls_body(hc_ref, w_ref, b_ref, out_ref):
    out_ref[...] = jnp.dot(hc_ref[...], w_ref[...]) + b_ref[...]


def _cls_call(hc, w, b):
    return pl.pallas_call(
        _cls_body,
        out_shape=jax.ShapeDtypeStruct((_N, _NCLS), jnp.float32),
    )(hc, w, b)


@functools.lru_cache(maxsize=None)
def _gather_kernel():
    mesh = plsc.VectorSubcoreMesh(core_axis_name="c", subcore_axis_name="s")

    @functools.partial(
        pl.kernel,
        mesh=mesh,
        out_type=jax.ShapeDtypeStruct((_E_PAD, _H), jnp.float32),
        scratch_types=[
            pltpu.VMEM((_NCHUNK, _CHUNK), jnp.int32),
            pltpu.VMEM((_EPW, _H), jnp.float32),
            pltpu.SemaphoreType.DMA,
        ],
        compiler_params=pltpu.CompilerParams(use_tc_tiling_on_sc=False),
    )
    def gather(tbl_hbm, src_hbm, out_hbm, idx_v, rows_v, sem):
        wid = lax.axis_index("s") * 2 + lax.axis_index("c")
        pltpu.sync_copy(src_hbm.at[wid], idx_v)

        def group(g, carry):
            base = g * _GRP
            copies = [
                pltpu.async_copy(
                    tbl_hbm.at[idx_v.at[base + b]],
                    rows_v.at[pl.ds((base + b) * _CHUNK, _CHUNK)],
                    sem,
                )
                for b in range(_GRP)
            ]
            for cp in copies:
                cp.wait()
            return carry

        lax.fori_loop(0, _NCHUNK // _GRP, group, 0, unroll=False)
        pltpu.sync_copy(rows_v, out_hbm.at[pl.ds(wid * _EPW, _EPW)])

    return gather


@functools.lru_cache(maxsize=None)
def _scatter_kernel():
    mesh = plsc.VectorSubcoreMesh(core_axis_name="c", subcore_axis_name="s")

    @functools.partial(
        pl.kernel,
        mesh=mesh,
        out_type=jax.ShapeDtypeStruct((2, _N_PAD, _H), jnp.float32),
        scratch_types=[
            pltpu.VMEM((_NCHUNK, _CHUNK), jnp.int32),
            pltpu.VMEM((_EPW, _H), jnp.float32),
            pltpu.VMEM_SHARED((_N_PAD, _H), jnp.float32),
            pltpu.SemaphoreType.DMA,
        ],
        compiler_params=pltpu.CompilerParams(use_tc_tiling_on_sc=False),
    )
    def scatter(vals_hbm, dst_hbm, z_hbm, out_hbm, idx_v, vals_v, acc_sh, sem):
        cid = lax.axis_index("c")
        sid = lax.axis_index("s")
        wid = sid * 2 + cid
        rows = pl.ds(sid * _RPT, _RPT)
        pltpu.sync_copy(z_hbm.at[rows], acc_sh.at[rows])
        pltpu.sync_copy(dst_hbm.at[wid], idx_v)
        pltpu.sync_copy(vals_hbm.at[pl.ds(wid * _EPW, _EPW)], vals_v)
        plsc.subcore_barrier()

        def chunk(j, carry):
            pltpu.sync_copy(vals_v.at[pl.ds(j * _CHUNK, _CHUNK)],
                            acc_sh.at[idx_v.at[j]], add=True)
            return carry

        lax.fori_loop(0, _NCHUNK, chunk, 0, unroll=False)
        plsc.subcore_barrier()
        pltpu.sync_copy(acc_sh.at[rows], out_hbm.at[cid, rows])

    return scatter


def kernel(x, edge_index, edge_attr, params):
    p = params
    f32 = jnp.float32
    pad = _E_PAD - _E
    src = edge_index[0].astype(jnp.int32)
    dst = edge_index[1].astype(jnp.int32)
    src3 = jnp.concatenate([src, jnp.zeros((pad,), jnp.int32)]) \
        .reshape(_NW, _NCHUNK, _CHUNK)
    dst3 = jnp.concatenate([dst, jnp.full((pad,), _N, jnp.int32)]) \
        .reshape(_NW, _NCHUNK, _CHUNK)
    zrows = jnp.zeros((_N_PAD, _H), f32)
    ones_vals = jnp.ones((_E_PAD, _H), f32)

    mean, var = _stats_call(edge_attr)
    a = p['fbn_g'] * lax.rsqrt(var + _EPS)
    c0 = p['fbn_b'] - mean * a
    w1f = a[:, None] * p['f1_W']
    b1f = (p['f1_b'] + c0 @ p['f1_W']).reshape(1, 32)
    ea_pad = jnp.concatenate([edge_attr, jnp.zeros((pad, _EDIM), f32)])
    w = _fnet_call(ea_pad, w1f, b1f,
                   p['f2_W'], p['f2_b'].reshape(1, 128),
                   p['f3_W'], p['f3_b'].reshape(1, _H * _H))
    wp = w.reshape(_E_PAD // 8, 8 * _H * _H)

    h = _encode_call(x, p['bn1_g'].reshape(1, _DIN), p['bn1_b'].reshape(1, _DIN),
                     p['enc_W'], p['enc_b'].reshape(1, _H),
                     p['bn2_g'].reshape(1, _H), p['bn2_b'].reshape(1, _H))

    cnt_acc = _scatter_kernel()(ones_vals, dst3, zrows)

    igw = p['ig_W']
    igb = p['ig_b'].reshape(1, _H)
    lig = p['lni_g'].reshape(1, _H)
    lib_ = p['lni_b'].reshape(1, _H)
    lhg = p['lnh_g'].reshape(1, _H)
    lhb = p['lnh_b'].reshape(1, _H)
    wih = p['gru_Wih'].T
    bih = p['gru_bih'].reshape(1, 3 * _H)
    whh = p['gru_Whh'].T
    bhh = p['gru_bhh'].reshape(1, 3 * _H)
    eccb = p['ecc_bias'].reshape(1, _H)

    history = [h]
    for _ in range(_ITERS):
        xj = _gather_kernel()(h, src3)
        msgp = _bmm_call(xj.reshape(_E_PAD // 8, 128), wp)
        acc = _scatter_kernel()(msgp.reshape(_E_PAD, _H), dst3, zrows)
        h = _gru_call(acc[0], acc[1], cnt_acc[0], cnt_acc[1], h,
                      igw, igb, lig, lib_, lhg, lhb, wih, bih, whh, bhh, eccb)
        history.append(h)

    hcat = jnp.concatenate(history, axis=1)
    return _cls_call(hcat, p['cls_W'], p['cls_b'].reshape(1, _NCLS))


# speedup vs baseline: 4.2696x; 1.2998x over previous
import functools

import jax
import jax.numpy as jnp
from jax import lax
from jax.experimental import pallas as pl
from jax.experimental.pallas import tpu as pltpu
from jax.experimental.pallas import tpu_sc as plsc

_N = 10000
_E = 160000
_DIN = 128
_H = 16
_EDIM = 16
_NCLS = 16
_ITERS = 4
_EPS = 1e-5

_NW = 32
_CHUNK = 128
_EPW = 5120
_NCHUNK = _EPW // _CHUNK
_E_PAD = _NW * _EPW
_N_PAD = 10240
_RPT = _N_PAD // 16
_GRP = 8

_FNET_T = 2048
_BMM_T = 4096


def _stats_body(ea_ref, s_ref, s2_ref):
    x = ea_ref[...]
    s_ref[...] = jnp.sum(x, axis=0, keepdims=True)
    s2_ref[...] = jnp.sum(x * x, axis=0, keepdims=True)


def _stats_call(ea):
    s, s2 = pl.pallas_call(
        _stats_body,
        out_shape=(jax.ShapeDtypeStruct((1, 128), jnp.float32),
                   jax.ShapeDtypeStruct((1, 128), jnp.float32)),
    )(ea.reshape(_E // 8, 128))
    m = s.reshape(8, _EDIM).sum(axis=0) / _E
    m2 = s2.reshape(8, _EDIM).sum(axis=0) / _E
    return m, m2 - m * m


def _fnet_body(e_ref, w1_ref, b1_ref, w2_ref, b2_ref, w3_ref, b3_ref, out_ref):
    eap = e_ref[...]
    cols = []
    for s in range(8):
        es = eap[:, s * _EDIM:(s + 1) * _EDIM]
        h1 = jnp.maximum(jnp.dot(es, w1_ref[...]) + b1_ref[...], 0.0)
        h2 = jnp.maximum(jnp.dot(h1, w2_ref[...]) + b2_ref[...], 0.0)
        cols.append(jnp.dot(h2, w3_ref[...]) + b3_ref[...])
    out_ref[...] = jnp.concatenate(cols, axis=1)


def _fnet_call(ea_p, w1, b1, w2, b2, w3, b3):
    t8 = _FNET_T // 8
    g = _E_PAD // _FNET_T
    fixed = lambda i: (0, 0)
    return pl.pallas_call(
        _fnet_body,
        grid=(g,),
        in_specs=[
            pl.BlockSpec((t8, 128), lambda i: (i, 0)),
            pl.BlockSpec((_EDIM, 32), fixed),
            pl.BlockSpec((1, 32), fixed),
            pl.BlockSpec((32, 128), fixed),
            pl.BlockSpec((1, 128), fixed),
            pl.BlockSpec((128, _H * _H), fixed),
            pl.BlockSpec((1, _H * _H), fixed),
        ],
        out_specs=pl.BlockSpec((t8, 8 * _H * _H), lambda i: (i, 0)),
        out_shape=jax.ShapeDtypeStruct((_E_PAD // 8, 8 * _H * _H), jnp.float32),
    )(ea_p, w1, b1, w2, b2, w3, b3)


def _encode_body(x_ref, g1_ref, b1_ref, ew_ref, eb_ref, g2_ref, b2_ref, out_ref):
    x = x_ref[...]
    m = jnp.mean(x, axis=0, keepdims=True)
    v = jnp.mean((x - m) * (x - m), axis=0, keepdims=True)
    h = (x - m) * lax.rsqrt(v + _EPS) * g1_ref[...] + b1_ref[...]
    h = jnp.dot(h, ew_ref[...]) + eb_ref[...]
    m2 = jnp.mean(h, axis=0, keepdims=True)
    v2 = jnp.mean((h - m2) * (h - m2), axis=0, keepdims=True)
    out_ref[...] = jnp.maximum(
        (h - m2) * lax.rsqrt(v2 + _EPS) * g2_ref[...] + b2_ref[...], 0.0)


def _encode_call(x, g1, b1, ew, eb, g2, b2):
    return pl.pallas_call(
        _encode_body,
        out_shape=jax.ShapeDtypeStruct((_N, _H), jnp.float32),
    )(x, g1, b1, ew, eb, g2, b2)


def _bmm_body(xj_ref, w_ref, out_ref):
    xjp = xj_ref[...]
    wp = w_ref[...]
    ri = lax.broadcasted_iota(jnp.int32, (_H, _H * _H), 0)
    ci = lax.broadcasted_iota(jnp.int32, (_H, _H * _H), 1)
    rep = (ci // _H == ri).astype(jnp.float32)
    si = lax.broadcasted_iota(jnp.int32, (_H * _H, _H), 0)
    so = lax.broadcasted_iota(jnp.int32, (_H * _H, _H), 1)
    ssum = (si % _H == so).astype(jnp.float32)
    cols = []
    for s in range(8):
        xs = xjp[:, s * _H:(s + 1) * _H]
        ws = wp[:, s * 256:(s + 1) * 256]
        cols.append(jnp.dot(jnp.dot(xs, rep) * ws, ssum))
    out_ref[...] = jnp.concatenate(cols, axis=1)


def _bmm_call(xjp, wp):
    g = _E_PAD // _BMM_T
    t8 = _BMM_T // 8
    return pl.pallas_call(
        _bmm_body,
        grid=(g,),
        in_specs=[
            pl.BlockSpec((t8, 128), lambda i: (i, 0)),
            pl.BlockSpec((t8, 8 * _H * _H), lambda i: (i, 0)),
        ],
        out_specs=pl.BlockSpec((t8, 128), lambda i: (i, 0)),
        out_shape=jax.ShapeDtypeStruct((_E_PAD // 8, 128), jnp.float32),
    )(xjp, wp)


def _sig(t):
    return 1.0 / (1.0 + jnp.exp(-t))


def _ln(v, g, b):
    m = jnp.mean(v, axis=-1, keepdims=True)
    var = jnp.mean((v - m) * (v - m), axis=-1, keepdims=True)
    return (v - m) * lax.rsqrt(var + _EPS) * g + b


def _gru_body(a0_ref, a1_ref, c0_ref, c1_ref, h_ref, igw_ref, igb_ref,
              lig_ref, lib_ref, lhg_ref, lhb_ref, wih_ref, bih_ref,
              whh_ref, bhh_ref, eccb_ref, out_ref):
    s = (a0_ref[...] + a1_ref[...])[:_N]
    cnt = (c0_ref[...] + c1_ref[...])[:_N, 0:1]
    msg = s / jnp.maximum(cnt, 1.0) + eccb_ref[...]
    h = h_ref[...]
    gate = _sig(jnp.dot(h, igw_ref[...]) + igb_ref[...])
    gated = _ln(gate * msg, lig_ref[...], lib_ref[...])
    h2 = _ln(h, lhg_ref[...], lhb_ref[...])
    gi = jnp.dot(gated, wih_ref[...]) + bih_ref[...]
    gh = jnp.dot(h2, whh_ref[...]) + bhh_ref[...]
    r = _sig(gi[:, :_H] + gh[:, :_H])
    z = _sig(gi[:, _H:2 * _H] + gh[:, _H:2 * _H])
    n = jnp.tanh(gi[:, 2 * _H:] + r * gh[:, 2 * _H:])
    out_ref[...] = (1.0 - z) * n + z * h2


def _gru_call(a0, a1, c0, c1, h, igw, igb, lig, lib_, lhg, lhb,
              wih, bih, whh, bhh, eccb):
    return pl.pallas_call(
        _gru_body,
        out_shape=jax.ShapeDtypeStruct((_N, _H), jnp.float32),
    )(a0, a1, c0, c1, h, igw, igb, lig, lib_, lhg, lhb, wih, bih, whh, bhh, eccb)


def _cls_body(hc_ref, w_ref, b_ref, out_ref):
    out_ref[...] = jnp.dot(hc_ref[...], w_ref[...]) + b_ref[...]


def _cls_call(hc, w, b):
    return pl.pallas_call(
        _cls_body,
        out_shape=jax.ShapeDtypeStruct((_N, _NCLS), jnp.float32),
    )(hc, w, b)


@functools.lru_cache(maxsize=None)
def _gather_kernel():
    mesh = plsc.VectorSubcoreMesh(core_axis_name="c", subcore_axis_name="s")

    @functools.partial(
        pl.kernel,
        mesh=mesh,
        out_type=jax.ShapeDtypeStruct((_E_PAD, _H), jnp.float32),
        scratch_types=[
            pltpu.VMEM((_NCHUNK, _CHUNK), jnp.int32),
            pltpu.VMEM((_EPW, _H), jnp.float32),
            pltpu.SemaphoreType.DMA,
        ],
        compiler_params=pltpu.CompilerParams(use_tc_tiling_on_sc=False),
    )
    def gather(tbl_hbm, src_hbm, out_hbm, idx_v, rows_v, sem):
        wid = lax.axis_index("s") * 2 + lax.axis_index("c")
        pltpu.sync_copy(src_hbm.at[wid], idx_v)

        def group(g, carry):
            base = g * _GRP
            copies = [
                pltpu.async_copy(
                    tbl_hbm.at[idx_v.at[base + b]],
                    rows_v.at[pl.ds((base + b) * _CHUNK, _CHUNK)],
                    sem,
                )
                for b in range(_GRP)
            ]
            for cp in copies:
                cp.wait()
            return carry

        lax.fori_loop(0, _NCHUNK // _GRP, group, 0, unroll=False)
        pltpu.sync_copy(rows_v, out_hbm.at[pl.ds(wid * _EPW, _EPW)])

    return gather


@functools.lru_cache(maxsize=None)
def _scatter_kernel():
    mesh = plsc.VectorSubcoreMesh(core_axis_name="c", subcore_axis_name="s")

    @functools.partial(
        pl.kernel,
        mesh=mesh,
        out_type=jax.ShapeDtypeStruct((2, _N_PAD, _H), jnp.float32),
        scratch_types=[
            pltpu.VMEM((_NCHUNK, _CHUNK), jnp.int32),
            pltpu.VMEM((_EPW, _H), jnp.float32),
            pltpu.VMEM_SHARED((_N_PAD, _H), jnp.float32),
            pltpu.SemaphoreType.DMA,
        ],
        compiler_params=pltpu.CompilerParams(use_tc_tiling_on_sc=False),
    )
    def scatter(vals_hbm, dst_hbm, z_hbm, out_hbm, idx_v, vals_v, acc_sh, sem):
        cid = lax.axis_index("c")
        sid = lax.axis_index("s")
        wid = sid * 2 + cid
        rows = pl.ds(sid * _RPT, _RPT)
        pltpu.sync_copy(z_hbm.at[rows], acc_sh.at[rows])
        pltpu.sync_copy(dst_hbm.at[wid], idx_v)
        pltpu.sync_copy(vals_hbm.at[pl.ds(wid * _EPW, _EPW)], vals_v)
        plsc.subcore_barrier()

        def chunk(j, carry):
            pltpu.sync_copy(vals_v.at[pl.ds(j * _CHUNK, _CHUNK)],
                            acc_sh.at[idx_v.at[j]], add=True)
            return carry

        lax.fori_loop(0, _NCHUNK, chunk, 0, unroll=False)
        plsc.subcore_barrier()
        pltpu.sync_copy(acc_sh.at[rows], out_hbm.at[cid, rows])

    return scatter


def kernel(x, edge_index, edge_attr, params):
    p = params
    f32 = jnp.float32
    pad = _E_PAD - _E
    src = edge_index[0].astype(jnp.int32)
    dst = edge_index[1].astype(jnp.int32)
    src3 = jnp.concatenate([src, jnp.zeros((pad,), jnp.int32)]) \
        .reshape(_NW, _NCHUNK, _CHUNK)
    dst3 = jnp.concatenate([dst, jnp.full((pad,), _N, jnp.int32)]) \
        .reshape(_NW, _NCHUNK, _CHUNK)
    zrows = jnp.zeros((_N_PAD, _H), f32)
    ones_vals = jnp.ones((_E_PAD // 8, 128), f32).reshape(_E_PAD, _H)

    mean, var = _stats_call(edge_attr)
    a = p['fbn_g'] * lax.rsqrt(var + _EPS)
    c0 = p['fbn_b'] - mean * a
    w1f = a[:, None] * p['f1_W']
    b1f = (p['f1_b'] + c0 @ p['f1_W']).reshape(1, 32)
    ea_p = jnp.concatenate([edge_attr.reshape(_E // 8, 128),
                            jnp.zeros((pad // 8, 128), f32)])
    wp = _fnet_call(ea_p, w1f, b1f,
                    p['f2_W'], p['f2_b'].reshape(1, 128),
                    p['f3_W'], p['f3_b'].reshape(1, _H * _H))

    h = _encode_call(x, p['bn1_g'].reshape(1, _DIN), p['bn1_b'].reshape(1, _DIN),
                     p['enc_W'], p['enc_b'].reshape(1, _H),
                     p['bn2_g'].reshape(1, _H), p['bn2_b'].reshape(1, _H))

    cnt_acc = _scatter_kernel()(ones_vals, dst3, zrows)

    igw = p['ig_W']
    igb = p['ig_b'].reshape(1, _H)
    lig = p['lni_g'].reshape(1, _H)
    lib_ = p['lni_b'].reshape(1, _H)
    lhg = p['lnh_g'].reshape(1, _H)
    lhb = p['lnh_b'].reshape(1, _H)
    wih = p['gru_Wih'].T
    bih = p['gru_bih'].reshape(1, 3 * _H)
    whh = p['gru_Whh'].T
    bhh = p['gru_bhh'].reshape(1, 3 * _H)
    eccb = p['ecc_bias'].reshape(1, _H)

    history = [h]
    for _ in range(_ITERS):
        xj = _gather_kernel()(h, src3)
        msgp = _bmm_call(xj.reshape(_E_PAD // 8, 128), wp)
        acc = _scatter_kernel()(msgp.reshape(_E_PAD, _H), dst3, zrows)
        h = _gru_call(acc[0], acc[1], cnt_acc[0], cnt_acc[1], h,
                      igw, igb, lig, lib_, lhg, lhb, wih, bih, whh, bhh, eccb)
        history.append(h)

    hcat = jnp.concatenate(history, axis=1)
    return _cls_call(hcat, p['cls_W'], p['cls_b'].reshape(1, _NCLS))
